# Initial kernel scaffold; baseline (speedup 1.0000x reference)
import jax
import jax.numpy as jnp
from jax.experimental import pallas as pl


def kernel(inputs, embeddings):
    raise NotImplementedError("write your pallas kernel here")


# speedup vs baseline: 1.1034x; 1.1034x over previous
import functools

import jax
import jax.numpy as jnp
from jax import lax
from jax.experimental import pallas as pl
from jax.experimental.pallas import tpu as pltpu
from jax.experimental.pallas import tpu_sc as plsc

_D = 64
_XFER = 128
_NUM_CORES = 2
_NUM_SUBCORES = 16
_NW = _NUM_CORES * _NUM_SUBCORES


@functools.lru_cache(maxsize=None)
def _build_gather(n_xfer_rows: int, d: int):
    xfers_per_worker = n_xfer_rows // _NW
    mesh = plsc.VectorSubcoreMesh(core_axis_name="c", subcore_axis_name="s")

    @functools.partial(
        pl.kernel,
        mesh=mesh,
        compiler_params=pltpu.CompilerParams(use_tc_tiling_on_sc=False),
        out_type=jax.ShapeDtypeStruct((n_xfer_rows * _XFER, d), jnp.float32),
        scratch_types=[
            pltpu.VMEM((xfers_per_worker, _XFER), jnp.int32),
            pltpu.VMEM((_XFER, d), jnp.float32),
            pltpu.SemaphoreType.DMA,
        ],
    )
    def gather(idx_hbm, table_hbm, out_hbm, idx_v, rows_v, sem):
        wid = lax.axis_index("s") * _NUM_CORES + lax.axis_index("c")
        row0 = wid * xfers_per_worker
        pltpu.sync_copy(idx_hbm.at[wid], idx_v)

        @pl.loop(0, xfers_per_worker)
        def _(j):
            pltpu.async_copy(table_hbm.at[idx_v.at[j]], rows_v, sem).wait()
            pltpu.sync_copy(rows_v, out_hbm.at[pl.ds((row0 + j) * _XFER, _XFER)])

    return gather


def kernel(inputs, embeddings):
    b, s = inputs.shape
    n = b * s
    d = embeddings.shape[1]
    idx3d = inputs.reshape(_NW, n // (_NW * _XFER), _XFER).astype(jnp.int32)
    out = _build_gather(n // _XFER, d)(idx3d, embeddings)
    return out.reshape(b, s, d)


# speedup vs baseline: 1.2154x; 1.1015x over previous
import functools

import jax
import jax.numpy as jnp
from jax import lax
from jax.experimental import pallas as pl
from jax.experimental.pallas import tpu as pltpu
from jax.experimental.pallas import tpu_sc as plsc

_D = 64
_XFER = 128
_NUM_CORES = 2
_NUM_SUBCORES = 16
_NW = _NUM_CORES * _NUM_SUBCORES
_K = 7


@functools.lru_cache(maxsize=None)
def _build_gather(n_xfer_rows: int, d: int):
    xpw = n_xfer_rows // _NW
    chunks = []
    s = 0
    while s < xpw:
        c = min(_K, xpw - s)
        chunks.append((s, c))
        s += c
    mesh = plsc.VectorSubcoreMesh(core_axis_name="c", subcore_axis_name="s")

    @functools.partial(
        pl.kernel,
        mesh=mesh,
        compiler_params=pltpu.CompilerParams(use_tc_tiling_on_sc=False),
        out_type=jax.ShapeDtypeStruct((n_xfer_rows * _XFER, d), jnp.float32),
        scratch_types=[
            pltpu.VMEM((xpw, _XFER), jnp.int32),
            pltpu.VMEM((2, _K * _XFER, d), jnp.float32),
            pltpu.SemaphoreType.DMA,
            pltpu.SemaphoreType.DMA,
        ],
    )
    def gather(idx_hbm, table_hbm, out_hbm, idx_v, rows_v, sem0, sem1):
        sems = (sem0, sem1)
        wid = lax.axis_index("s") * _NUM_CORES + lax.axis_index("c")
        row0 = wid * xpw * _XFER
        pltpu.sync_copy(idx_hbm.at[wid], idx_v)

        def fire(ci):
            s0, c = chunks[ci]
            buf = ci % 2
            return [
                pltpu.async_copy(
                    table_hbm.at[idx_v.at[s0 + j]],
                    rows_v.at[buf, pl.ds(j * _XFER, _XFER)],
                    sems[buf],
                )
                for j in range(c)
            ]

        pending = fire(0)
        for ci in range(len(chunks)):
            nxt = fire(ci + 1) if ci + 1 < len(chunks) else []
            for cp in pending:
                cp.wait()
            s0, c = chunks[ci]
            pltpu.sync_copy(
                rows_v.at[ci % 2, pl.ds(0, c * _XFER)],
                out_hbm.at[pl.ds(row0 + s0 * _XFER, c * _XFER)],
            )
            pending = nxt

    return gather


def kernel(inputs, embeddings):
    b, s = inputs.shape
    n = b * s
    d = embeddings.shape[1]
    idx3d = inputs.reshape(_NW, n // (_NW * _XFER), _XFER).astype(jnp.int32)
    out = _build_gather(n // _XFER, d)(idx3d, embeddings)
    return out.reshape(b, s, d)
